# Initial kernel scaffold; baseline (speedup 1.0000x reference)
import jax
import jax.numpy as jnp
from jax.experimental import pallas as pl


def kernel(query_points, key_points, g_in, b_in, W_qkv, W_o, b_o, W_knn1, b_knn1, W_knn2, b_knn2, W_sm, b_sm, W_cm, b_cm, g_cq, b_cq, g_ck, b_ck, W_cq, W_ck, W_cv, g_ff, b_ff, W_ff1, b_ff1, W_ff2, b_ff2):
    raise NotImplementedError("write your pallas kernel here")


# speedup vs baseline: 6.2460x; 6.2460x over previous
import functools

import jax
import jax.numpy as jnp
import numpy as np
from jax import lax
from jax.experimental import pallas as pl
from jax.experimental.pallas import tpu as pltpu
from jax.experimental.pallas import tpu_sc as plsc

D = 384
H = 6
DH = D // H
KNN = 8
N = 2048
NBLK = 256
QBLK = 512


def _layernorm(x, g, b):
    m = jnp.mean(x, axis=-1, keepdims=True)
    xc = x - m
    v = jnp.mean(xc * xc, axis=-1, keepdims=True)
    return xc * lax.rsqrt(v + 1e-5) * g + b


def _lrelu(x):
    return jnp.where(x >= 0, x, 0.2 * x)


def _prep_q_body(qf_ref, g_ref, b_ref, wqkv_ref, wk1_ref, bk1_ref,
                 qkv_ref, p1_ref, cc1_ref):
    nf = _layernorm(qf_ref[0], g_ref[0], b_ref[0])
    qkv_ref[0] = jnp.dot(nf, wqkv_ref[...], preferred_element_type=jnp.float32)
    w1a = wk1_ref[:D]
    w1b = wk1_ref[D:]
    p1_ref[0] = jnp.dot(nf, w1a, preferred_element_type=jnp.float32)
    cc1_ref[0] = jnp.dot(nf, w1b - w1a, preferred_element_type=jnp.float32) + bk1_ref[0]


def _prep_q(qf_t, g_in, b_in, W_qkv, W_knn1, b_knn1):
    B = qf_t.shape[0]
    grid = (B, N // NBLK)
    return pl.pallas_call(
        _prep_q_body,
        grid=grid,
        in_specs=[
            pl.BlockSpec((1, NBLK, D), lambda b, i: (b, i, 0)),
            pl.BlockSpec((1, D), lambda b, i: (0, 0)),
            pl.BlockSpec((1, D), lambda b, i: (0, 0)),
            pl.BlockSpec((D, 3 * D), lambda b, i: (0, 0)),
            pl.BlockSpec((2 * D, D), lambda b, i: (0, 0)),
            pl.BlockSpec((1, D), lambda b, i: (0, 0)),
        ],
        out_specs=[
            pl.BlockSpec((1, NBLK, 3 * D), lambda b, i: (b, i, 0)),
            pl.BlockSpec((1, NBLK, D), lambda b, i: (b, i, 0)),
            pl.BlockSpec((1, NBLK, D), lambda b, i: (b, i, 0)),
        ],
        out_shape=[
            jax.ShapeDtypeStruct((B, N, 3 * D), jnp.float32),
            jax.ShapeDtypeStruct((B, N, D), jnp.float32),
            jax.ShapeDtypeStruct((B, N, D), jnp.float32),
        ],
    )(qf_t, g_in, b_in, W_qkv, W_knn1, b_knn1)


def _prep_k_body(kf_ref, g_ref, b_ref, wck_ref, wcv_ref, wk2_ref,
                 ck_ref, cv_ref, p2_ref):
    nk = _layernorm(kf_ref[0], g_ref[0], b_ref[0])
    ck_ref[0] = jnp.dot(nk, wck_ref[...], preferred_element_type=jnp.float32)
    cv_ref[0] = jnp.dot(nk, wcv_ref[...], preferred_element_type=jnp.float32)
    p2_ref[0] = jnp.dot(nk, wk2_ref[:D], preferred_element_type=jnp.float32)


def _prep_k(kf_t, g_ck, b_ck, W_ck, W_cv, W_knn2):
    B = kf_t.shape[0]
    grid = (B, N // NBLK)
    return pl.pallas_call(
        _prep_k_body,
        grid=grid,
        in_specs=[
            pl.BlockSpec((1, NBLK, D), lambda b, i: (b, i, 0)),
            pl.BlockSpec((1, D), lambda b, i: (0, 0)),
            pl.BlockSpec((1, D), lambda b, i: (0, 0)),
            pl.BlockSpec((D, D), lambda b, i: (0, 0)),
            pl.BlockSpec((D, D), lambda b, i: (0, 0)),
            pl.BlockSpec((2 * D, D), lambda b, i: (0, 0)),
        ],
        out_specs=[
            pl.BlockSpec((1, NBLK, D), lambda b, i: (b, i, 0)),
            pl.BlockSpec((1, NBLK, D), lambda b, i: (b, i, 0)),
            pl.BlockSpec((1, NBLK, D), lambda b, i: (b, i, 0)),
        ],
        out_shape=[
            jax.ShapeDtypeStruct((B, N, D), jnp.float32),
            jax.ShapeDtypeStruct((B, N, D), jnp.float32),
            jax.ShapeDtypeStruct((B, N, D), jnp.float32),
        ],
    )(kf_t, g_ck, b_ck, W_ck, W_cv, W_knn2)


def _mha_body(q_ref, k_ref, v_ref, o_ref):
    q = q_ref[0, 0]
    k = k_ref[0, 0]
    v = v_ref[0, 0]
    s = lax.dot_general(q, k, (((1,), (1,)), ((), ())),
                        preferred_element_type=jnp.float32) * (1.0 / np.sqrt(DH))
    m = jnp.max(s, axis=-1, keepdims=True)
    e = jnp.exp(s - m)
    p = e / jnp.sum(e, axis=-1, keepdims=True)
    o_ref[0, 0] = jnp.dot(p, v, preferred_element_type=jnp.float32)


def _split_heads(x):
    B = x.shape[0]
    return x.reshape(B, N, H, DH).transpose(0, 2, 1, 3)


def _merge_heads(x):
    B = x.shape[0]
    return x.transpose(0, 2, 1, 3).reshape(B, N, D)


def _mha(q, k, v):
    B = q.shape[0]
    grid = (B, H, N // QBLK)
    return pl.pallas_call(
        _mha_body,
        grid=grid,
        in_specs=[
            pl.BlockSpec((1, 1, QBLK, DH), lambda b, h, i: (b, h, i, 0)),
            pl.BlockSpec((1, 1, N, DH), lambda b, h, i: (b, h, 0, 0)),
            pl.BlockSpec((1, 1, N, DH), lambda b, h, i: (b, h, 0, 0)),
        ],
        out_specs=pl.BlockSpec((1, 1, QBLK, DH), lambda b, h, i: (b, h, i, 0)),
        out_shape=jax.ShapeDtypeStruct((B, H, N, DH), jnp.float32),
    )(q, k, v)


def _knn_body(qb_ref, qfull_ref, kfull_ref, i1_ref, i2_ref):
    b = pl.program_id(0)
    qb = qb_ref[0]
    qn = jnp.sum(qb * qb, axis=-1, keepdims=True)

    def top8(keys):
        kn = jnp.sum(keys * keys, axis=-1)
        d = (qn + kn[None, :]
             - 2.0 * lax.dot_general(qb, keys, (((1,), (1,)), ((), ())),
                                     preferred_element_type=jnp.float32))
        cols = lax.broadcasted_iota(jnp.int32, d.shape, 1)
        outs = []
        for _ in range(KNN):
            mval = jnp.min(d, axis=-1, keepdims=True)
            ij = jnp.min(jnp.where(d <= mval, cols, N), axis=-1)
            outs.append(ij.reshape(1, NBLK))
            d = jnp.where(cols == ij[:, None], jnp.float32(3.0e38), d)
        return jnp.concatenate(outs, axis=0)

    i1_ref[0] = top8(qfull_ref[0]) + b * N
    i2_ref[0] = top8(kfull_ref[0]) + b * N


def _knn(qc_p, kc_p):
    B = qc_p.shape[0]
    grid = (B, N // NBLK)
    return pl.pallas_call(
        _knn_body,
        grid=grid,
        in_specs=[
            pl.BlockSpec((1, NBLK, 8), lambda b, i: (b, i, 0)),
            pl.BlockSpec((1, N, 8), lambda b, i: (b, 0, 0)),
            pl.BlockSpec((1, N, 8), lambda b, i: (b, 0, 0)),
        ],
        out_specs=[
            pl.BlockSpec((1, KNN, NBLK), lambda b, i: (b, 0, i)),
            pl.BlockSpec((1, KNN, NBLK), lambda b, i: (b, 0, i)),
        ],
        out_shape=[
            jax.ShapeDtypeStruct((B, KNN, N), jnp.int32),
            jax.ShapeDtypeStruct((B, KNN, N), jnp.int32),
        ],
    )(qc_p, qc_p, kc_p)


def _gather_rows(table, idx_flat):
    num_rows = idx_flat.shape[0]
    info = plsc.get_sparse_core_info()
    nw = info.num_cores * info.num_subcores
    per_w = num_rows // nw
    chunk = 128
    nchunks = per_w // chunk
    mesh = plsc.VectorSubcoreMesh(core_axis_name="c", subcore_axis_name="s")

    @functools.partial(
        pl.kernel,
        mesh=mesh,
        out_type=jax.ShapeDtypeStruct((num_rows, D), jnp.float32),
        scratch_types=[
            pltpu.VMEM((chunk,), jnp.int32),
            pltpu.VMEM((chunk, D), jnp.float32),
            pltpu.SemaphoreType.DMA,
        ],
    )
    def gk(table_hbm, idx_hbm, out_hbm, idx_v, rows_v, sem):
        wid = lax.axis_index("s") * info.num_cores + lax.axis_index("c")
        base = wid * per_w
        for c in range(nchunks):
            off = base + c * chunk
            pltpu.sync_copy(idx_hbm.at[pl.ds(off, chunk)], idx_v)
            pltpu.async_copy(table_hbm.at[idx_v], rows_v, sem).wait()
            pltpu.sync_copy(rows_v, out_hbm.at[pl.ds(off, chunk)])

    return gk(table, idx_flat)


def _comb1_body(h_ref, g1_ref, cc1_ref, qf_ref, wo_ref, bo_ref, wsm_ref,
                bsm_ref, gcq_ref, bcq_ref, wcq_ref, wk2_ref, bk2_ref,
                qfeat_ref, cq_ref, cc2_ref):
    sa = jnp.dot(h_ref[0], wo_ref[...], preferred_element_type=jnp.float32) + bo_ref[0]
    g = g1_ref[...]
    cc = cc1_ref[0]
    acc = _lrelu(g[0] + cc)
    for j in range(1, KNN):
        acc = jnp.maximum(acc, _lrelu(g[j] + cc))
    out1 = (jnp.dot(sa, wsm_ref[:D], preferred_element_type=jnp.float32)
            + jnp.dot(acc, wsm_ref[D:], preferred_element_type=jnp.float32)
            + bsm_ref[0])
    qfeat = out1 + qf_ref[0]
    qfeat_ref[0] = qfeat
    nq = _layernorm(qfeat, gcq_ref[0], bcq_ref[0])
    cq_ref[0] = jnp.dot(nq, wcq_ref[...], preferred_element_type=jnp.float32)
    w2a = wk2_ref[:D]
    w2b = wk2_ref[D:]
    cc2_ref[0] = jnp.dot(nq, w2b - w2a, preferred_element_type=jnp.float32) + bk2_ref[0]


def _comb1(heads, G1, Cc1, qf_t, W_o, b_o, W_sm, b_sm, g_cq, b_cq, W_cq,
           W_knn2, b_knn2):
    B = heads.shape[0]
    grid = (B, N // NBLK)
    blk = lambda b, i: (b, i, 0)
    wfull = lambda b, i: (0, 0)
    return pl.pallas_call(
        _comb1_body,
        grid=grid,
        in_specs=[
            pl.BlockSpec((1, NBLK, D), blk),
            pl.BlockSpec((KNN, NBLK, D), lambda b, i: (0, b * (N // NBLK) + i, 0)),
            pl.BlockSpec((1, NBLK, D), blk),
            pl.BlockSpec((1, NBLK, D), blk),
            pl.BlockSpec((D, D), wfull),
            pl.BlockSpec((1, D), wfull),
            pl.BlockSpec((2 * D, D), wfull),
            pl.BlockSpec((1, D), wfull),
            pl.BlockSpec((1, D), wfull),
            pl.BlockSpec((1, D), wfull),
            pl.BlockSpec((D, D), wfull),
            pl.BlockSpec((2 * D, D), wfull),
            pl.BlockSpec((1, D), wfull),
        ],
        out_specs=[
            pl.BlockSpec((1, NBLK, D), blk),
            pl.BlockSpec((1, NBLK, D), blk),
            pl.BlockSpec((1, NBLK, D), blk),
        ],
        out_shape=[
            jax.ShapeDtypeStruct((B, N, D), jnp.float32),
            jax.ShapeDtypeStruct((B, N, D), jnp.float32),
            jax.ShapeDtypeStruct((B, N, D), jnp.float32),
        ],
    )(heads, G1, Cc1, qf_t, W_o, b_o, W_sm, b_sm, g_cq, b_cq, W_cq, W_knn2,
      b_knn2)


def _comb2_body(h_ref, g2_ref, cc2_ref, qfeat_ref, wo_ref, bo_ref, wcm_ref,
                bcm_ref, gff_ref, bff_ref, wff1_ref, bff1_ref, wff2_ref,
                bff2_ref, out_ref):
    cr = jnp.dot(h_ref[0], wo_ref[...], preferred_element_type=jnp.float32) + bo_ref[0]
    g = g2_ref[...]
    cc = cc2_ref[0]
    acc = _lrelu(g[0] + cc)
    for j in range(1, KNN):
        acc = jnp.maximum(acc, _lrelu(g[j] + cc))
    out2 = (jnp.dot(cr, wcm_ref[:D], preferred_element_type=jnp.float32)
            + jnp.dot(acc, wcm_ref[D:], preferred_element_type=jnp.float32)
            + bcm_ref[0])
    qf2 = qfeat_ref[0] + out2
    nff = _layernorm(qf2, gff_ref[0], bff_ref[0])
    ff = jnp.maximum(
        jnp.dot(nff, wff1_ref[...], preferred_element_type=jnp.float32) + bff1_ref[0],
        0.0)
    ff = jnp.dot(ff, wff2_ref[...], preferred_element_type=jnp.float32) + bff2_ref[0]
    out_ref[0] = qf2 + ff


def _comb2(heads, G2, Cc2, qfeat, W_o, b_o, W_cm, b_cm, g_ff, b_ff, W_ff1,
           b_ff1, W_ff2, b_ff2):
    B = heads.shape[0]
    grid = (B, N // NBLK)
    blk = lambda b, i: (b, i, 0)
    wfull = lambda b, i: (0, 0)
    return pl.pallas_call(
        _comb2_body,
        grid=grid,
        in_specs=[
            pl.BlockSpec((1, NBLK, D), blk),
            pl.BlockSpec((KNN, NBLK, D), lambda b, i: (0, b * (N // NBLK) + i, 0)),
            pl.BlockSpec((1, NBLK, D), blk),
            pl.BlockSpec((1, NBLK, D), blk),
            pl.BlockSpec((D, D), wfull),
            pl.BlockSpec((1, D), wfull),
            pl.BlockSpec((2 * D, D), wfull),
            pl.BlockSpec((1, D), wfull),
            pl.BlockSpec((1, D), wfull),
            pl.BlockSpec((1, D), wfull),
            pl.BlockSpec((D, 2 * D), wfull),
            pl.BlockSpec((1, 2 * D), wfull),
            pl.BlockSpec((2 * D, D), wfull),
            pl.BlockSpec((1, D), wfull),
        ],
        out_specs=pl.BlockSpec((1, NBLK, D), blk),
        out_shape=jax.ShapeDtypeStruct((B, N, D), jnp.float32),
    )(heads, G2, Cc2, qfeat, W_o, b_o, W_cm, b_cm, g_ff, b_ff, W_ff1, b_ff1,
      W_ff2, b_ff2)


def kernel(query_points, key_points, g_in, b_in, W_qkv, W_o, b_o, W_knn1,
           b_knn1, W_knn2, b_knn2, W_sm, b_sm, W_cm, b_cm, g_cq, b_cq, g_ck,
           b_ck, W_cq, W_ck, W_cv, g_ff, b_ff, W_ff1, b_ff1, W_ff2, b_ff2):
    B = query_points.shape[0]
    f32 = jnp.float32
    qc = query_points[:, :3, :]
    kc = key_points[:, :3, :]
    qf_t = query_points[:, 3:, :].transpose(0, 2, 1)
    kf_t = key_points[:, 3:, :].transpose(0, 2, 1)

    zpad = jnp.zeros((B, N, 5), f32)
    qc_p = jnp.concatenate([qc.transpose(0, 2, 1), zpad], axis=-1)
    kc_p = jnp.concatenate([kc.transpose(0, 2, 1), zpad], axis=-1)

    r2 = lambda x: x.reshape(1, -1)
    qkv, P1, Cc1 = _prep_q(qf_t, r2(g_in), r2(b_in), W_qkv, W_knn1, r2(b_knn1))
    ck, cv, P2 = _prep_k(kf_t, r2(g_ck), r2(b_ck), W_ck, W_cv, W_knn2)

    q = _split_heads(qkv[..., :D])
    k = _split_heads(qkv[..., D:2 * D])
    v = _split_heads(qkv[..., 2 * D:])
    heads1 = _merge_heads(_mha(q, k, v))

    idx1, idx2 = _knn(qc_p, kc_p)
    i1_flat = idx1.transpose(1, 0, 2).reshape(-1)
    i2_flat = idx2.transpose(1, 0, 2).reshape(-1)

    G1 = _gather_rows(P1.reshape(B * N, D), i1_flat).reshape(KNN, B * N, D)

    qfeat, cq, Cc2 = _comb1(heads1, G1, Cc1, qf_t, W_o, r2(b_o), W_sm,
                            r2(b_sm), r2(g_cq), r2(b_cq), W_cq, W_knn2,
                            r2(b_knn2))

    heads2 = _merge_heads(_mha(_split_heads(cq), _split_heads(ck),
                               _split_heads(cv)))
    G2 = _gather_rows(P2.reshape(B * N, D), i2_flat).reshape(KNN, B * N, D)

    out = _comb2(heads2, G2, Cc2, qfeat, W_o, r2(b_o), W_cm, r2(b_cm),
                 r2(g_ff), r2(b_ff), W_ff1, r2(b_ff1), W_ff2, r2(b_ff2))

    return jnp.concatenate([qc, out.transpose(0, 2, 1)], axis=1)
